# baseline (speedup 1.0000x reference)
import functools

import jax
import jax.numpy as jnp
from jax import lax
from jax.experimental import pallas as pl
from jax.experimental.pallas import tpu as pltpu
from jax.experimental.pallas import tpu_sc as plsc

_NC = 2
_NS = 16
_NW = _NC * _NS
_C = 64


def _gather_sum(table, neigh_c, npad):
    nq, s, c = neigh_c.shape
    d = table.shape[1]
    k0 = (nq * 13) // 16
    mesh = plsc.VectorSubcoreMesh(core_axis_name="c", subcore_axis_name="s")

    @functools.partial(
        pl.kernel,
        out_type=jax.ShapeDtypeStruct((npad, d), jnp.float32),
        mesh=mesh,
        scratch_types=[
            pltpu.VMEM((s, _C), jnp.int32),
            pltpu.VMEM((_C, d), jnp.float32),
            pltpu.SemaphoreType.DMA,
            pltpu.SemaphoreType.DMA,
        ],
    )
    def gather_kernel(table_hbm, neigh_hbm, out_hbm, idx_v, acc_v, sem0, sem):
        cid = lax.axis_index("c")
        sid = lax.axis_index("s")
        k1 = nq - k0
        lo = jnp.where(cid == 0, (sid * k0) // _NS, k0 + (sid * k1) // _NS)
        hi = jnp.where(cid == 0, ((sid + 1) * k0) // _NS,
                       k0 + ((sid + 1) * k1) // _NS)

        def chunk(q, carry):
            base = q * _C
            pltpu.sync_copy(neigh_hbm.at[q], idx_v)
            pltpu.async_copy(table_hbm.at[idx_v.at[0]], acc_v, sem0).wait()

            def fire(j, cy):
                pltpu.async_copy(table_hbm.at[idx_v.at[j]], acc_v, sem,
                                 add=True)
                return cy

            lax.fori_loop(1, s, fire, 0)

            def drain(j, cy):
                pltpu.make_async_copy(table_hbm.at[idx_v.at[0]], acc_v,
                                      sem).wait()
                return cy

            lax.fori_loop(1, s, drain, 0)
            pltpu.sync_copy(acc_v, out_hbm.at[pl.ds(base, _C)])
            return carry

        lax.fori_loop(lo, hi, chunk, 0)

    return gather_kernel(table, neigh_c)


def _matmul(x, w):
    def body(x_ref, w_ref, o_ref):
        o_ref[...] = jnp.dot(x_ref[...], w_ref[...],
                             preferred_element_type=jnp.float32)

    return pl.pallas_call(
        body,
        out_shape=jax.ShapeDtypeStruct((x.shape[0], w.shape[1]), jnp.float32),
    )(x, w)


def _layer0(x, agg_sum, w_self, w_neigh, scale):
    def body(x_ref, s_ref, wa_ref, wb_ref, o_ref):
        m = jnp.dot(x_ref[...], wa_ref[...], preferred_element_type=jnp.float32)
        m = m + jnp.dot(s_ref[...] * scale, wb_ref[...],
                        preferred_element_type=jnp.float32)
        o_ref[...] = jnp.maximum(m, 0.0)

    return pl.pallas_call(
        body,
        out_shape=jax.ShapeDtypeStruct((x.shape[0], w_self.shape[1]), jnp.float32),
    )(x, agg_sum, w_self, w_neigh)


def _layer1(h0, agg_sum, w_self, w_neigh, scale):
    h = h0.shape[1]

    def body(h_ref, s_ref, wa_ref, wb_ref, o_ref):
        m = jnp.dot(h_ref[...], wa_ref[...], preferred_element_type=jnp.float32)
        m = m + jnp.dot(s_ref[...] * scale, wb_ref[...],
                        preferred_element_type=jnp.float32)
        o_ref[:, :h] = h_ref[...]
        o_ref[:, h:] = jnp.maximum(m, 0.0)

    return pl.pallas_call(
        body,
        out_shape=jax.ShapeDtypeStruct(
            (h0.shape[0], h + w_self.shape[1]), jnp.float32),
    )(h0, agg_sum, w_self, w_neigh)


def kernel(feats, node_neigh, W_prep, W0, W1):
    n, s = node_neigh.shape
    p = W_prep.shape[1]
    h0_dim = W0.shape[1]
    scale = 1.0 / s

    npad = -(-n // (_NW * _C)) * (_NW * _C)
    neigh_t = jnp.pad(node_neigh.T, ((0, 0), (0, npad - n)))
    neigh_c = neigh_t.reshape(s, npad // _C, _C).transpose(1, 0, 2)

    all_feats = _matmul(feats, W_prep)
    s0 = _gather_sum(all_feats, neigh_c, npad)[:n]
    h0 = _layer0(all_feats, s0, W0[:p], W0[p:], scale)
    s1 = _gather_sum(h0, neigh_c, npad)[:n]
    out = _layer1(h0, s1, W1[:h0_dim], W1[h0_dim:], scale)
    return out[None]


# speedup vs baseline: 1.0015x; 1.0015x over previous
import functools

import jax
import jax.numpy as jnp
from jax import lax
from jax.experimental import pallas as pl
from jax.experimental.pallas import tpu as pltpu
from jax.experimental.pallas import tpu_sc as plsc

_NC = 2
_NS = 16
_NW = _NC * _NS
_C = 64


def _gather_sum(table, neigh_c, npad):
    nq, s, c = neigh_c.shape
    d = table.shape[1]
    k0 = (nq * 13) // 16
    mesh = plsc.VectorSubcoreMesh(core_axis_name="c", subcore_axis_name="s")

    @functools.partial(
        pl.kernel,
        out_type=jax.ShapeDtypeStruct((npad, d), jnp.float32),
        mesh=mesh,
        scratch_types=[
            pltpu.VMEM((s, _C), jnp.int32),
            pltpu.VMEM((_C, d), jnp.float32),
            pltpu.SemaphoreType.DMA,
            pltpu.SemaphoreType.DMA,
        ],
    )
    def gather_kernel(table_hbm, neigh_hbm, out_hbm, idx_v, acc_v, sem0, sem):
        cid = lax.axis_index("c")
        sid = lax.axis_index("s")
        k1 = nq - k0
        lo = jnp.where(cid == 0, (sid * k0) // _NS, k0 + (sid * k1) // _NS)
        hi = jnp.where(cid == 0, ((sid + 1) * k0) // _NS,
                       k0 + ((sid + 1) * k1) // _NS)

        def chunk(q, carry):
            base = q * _C
            with jax.named_scope("idx_load"):
                pltpu.sync_copy(neigh_hbm.at[q], idx_v)
            with jax.named_scope("first_gather"):
                pltpu.async_copy(table_hbm.at[idx_v.at[0]], acc_v, sem0).wait()

            def fire(j, cy):
                pltpu.async_copy(table_hbm.at[idx_v.at[j]], acc_v, sem,
                                 add=True)
                return cy

            with jax.named_scope("fire"):
                lax.fori_loop(1, s, fire, 0)

            def drain(j, cy):
                pltpu.make_async_copy(table_hbm.at[idx_v.at[0]], acc_v,
                                      sem).wait()
                return cy

            with jax.named_scope("drain"):
                lax.fori_loop(1, s, drain, 0)
            with jax.named_scope("out_copy"):
                pltpu.sync_copy(acc_v, out_hbm.at[pl.ds(base, _C)])
            return carry

        lax.fori_loop(lo, hi, chunk, 0)

    return gather_kernel(table, neigh_c)


def _matmul(x, w):
    def body(x_ref, w_ref, o_ref):
        o_ref[...] = jnp.dot(x_ref[...], w_ref[...],
                             preferred_element_type=jnp.float32)

    return pl.pallas_call(
        body,
        out_shape=jax.ShapeDtypeStruct((x.shape[0], w.shape[1]), jnp.float32),
    )(x, w)


def _layer0(x, agg_sum, w_self, w_neigh, scale):
    def body(x_ref, s_ref, wa_ref, wb_ref, o_ref):
        m = jnp.dot(x_ref[...], wa_ref[...], preferred_element_type=jnp.float32)
        m = m + jnp.dot(s_ref[...] * scale, wb_ref[...],
                        preferred_element_type=jnp.float32)
        o_ref[...] = jnp.maximum(m, 0.0)

    return pl.pallas_call(
        body,
        out_shape=jax.ShapeDtypeStruct((x.shape[0], w_self.shape[1]), jnp.float32),
    )(x, agg_sum, w_self, w_neigh)


def _layer1(h0, agg_sum, w_self, w_neigh, scale):
    h = h0.shape[1]

    def body(h_ref, s_ref, wa_ref, wb_ref, o_ref):
        m = jnp.dot(h_ref[...], wa_ref[...], preferred_element_type=jnp.float32)
        m = m + jnp.dot(s_ref[...] * scale, wb_ref[...],
                        preferred_element_type=jnp.float32)
        o_ref[:, :h] = h_ref[...]
        o_ref[:, h:] = jnp.maximum(m, 0.0)

    return pl.pallas_call(
        body,
        out_shape=jax.ShapeDtypeStruct(
            (h0.shape[0], h + w_self.shape[1]), jnp.float32),
    )(h0, agg_sum, w_self, w_neigh)


def kernel(feats, node_neigh, W_prep, W0, W1):
    n, s = node_neigh.shape
    p = W_prep.shape[1]
    h0_dim = W0.shape[1]
    scale = 1.0 / s

    npad = -(-n // (_NW * _C)) * (_NW * _C)
    neigh_t = jnp.pad(node_neigh.T, ((0, 0), (0, npad - n)))
    neigh_c = neigh_t.reshape(s, npad // _C, _C).transpose(1, 0, 2)

    all_feats = _matmul(feats, W_prep)
    s0 = _gather_sum(all_feats, neigh_c, npad)[:n]
    h0 = _layer0(all_feats, s0, W0[:p], W0[p:], scale)
    s1 = _gather_sum(h0, neigh_c, npad)[:n]
    out = _layer1(h0, s1, W1[:h0_dim], W1[h0_dim:], scale)
    return out[None]


# speedup vs baseline: 4.0509x; 4.0449x over previous
import functools

import jax
import jax.numpy as jnp
from jax import lax
from jax.experimental import pallas as pl
from jax.experimental.pallas import tpu as pltpu
from jax.experimental.pallas import tpu_sc as plsc

_NC = 2
_NS = 16
_NW = _NC * _NS
_C = 64


def _gather_sum(table, neigh_c, npad):
    nq, s, c = neigh_c.shape
    d = table.shape[1]
    k0 = nq // 2
    mesh = plsc.VectorSubcoreMesh(core_axis_name="c", subcore_axis_name="s")

    @functools.partial(
        pl.kernel,
        out_type=jax.ShapeDtypeStruct((npad, d), jnp.float32),
        mesh=mesh,
        scratch_types=[
            pltpu.VMEM((s, _C), jnp.int32),
            pltpu.VMEM((_C, d), jnp.float32),
            pltpu.SemaphoreType.DMA,
            pltpu.SemaphoreType.DMA,
        ],
    )
    def gather_kernel(table_hbm, neigh_hbm, out_hbm, idx_v, acc_v, sem0, sem):
        cid = lax.axis_index("c")
        sid = lax.axis_index("s")
        k1 = nq - k0
        lo = jnp.where(cid == 0, (sid * k0) // _NS, k0 + (sid * k1) // _NS)
        hi = jnp.where(cid == 0, ((sid + 1) * k0) // _NS,
                       k0 + ((sid + 1) * k1) // _NS)

        def chunk(q, carry):
            base = q * _C
            with jax.named_scope("idx_load"):
                pltpu.sync_copy(neigh_hbm.at[q], idx_v)
            with jax.named_scope("first_gather"):
                pltpu.async_copy(table_hbm.at[idx_v.at[0]], acc_v, sem0).wait()

            def fire(j, cy):
                pltpu.async_copy(table_hbm.at[idx_v.at[j]], acc_v, sem,
                                 add=True)
                return cy

            with jax.named_scope("fire"):
                lax.fori_loop(1, s, fire, 0)

            def drain(j, cy):
                pltpu.make_async_copy(table_hbm.at[idx_v.at[0]], acc_v,
                                      sem).wait()
                return cy

            with jax.named_scope("drain"):
                lax.fori_loop(1, s, drain, 0)
            with jax.named_scope("out_copy"):
                pltpu.sync_copy(acc_v, out_hbm.at[pl.ds(base, _C)])
            return carry

        lax.fori_loop(lo, hi, chunk, 0)

    return gather_kernel(table, neigh_c)


def _matmul(x, w):
    def body(x_ref, w_ref, o_ref):
        o_ref[...] = jnp.dot(x_ref[...], w_ref[...],
                             preferred_element_type=jnp.float32)

    return pl.pallas_call(
        body,
        out_shape=jax.ShapeDtypeStruct((x.shape[0], w.shape[1]), jnp.float32),
    )(x, w)


def _layer0(x, agg_sum, w_self, w_neigh, scale):
    def body(x_ref, s_ref, wa_ref, wb_ref, o_ref):
        m = jnp.dot(x_ref[...], wa_ref[...], preferred_element_type=jnp.float32)
        m = m + jnp.dot(s_ref[...] * scale, wb_ref[...],
                        preferred_element_type=jnp.float32)
        o_ref[...] = jnp.maximum(m, 0.0)

    return pl.pallas_call(
        body,
        out_shape=jax.ShapeDtypeStruct((x.shape[0], w_self.shape[1]), jnp.float32),
    )(x, agg_sum, w_self, w_neigh)


def _layer1(h0, agg_sum, w_self, w_neigh, scale):
    h = h0.shape[1]

    def body(h_ref, s_ref, wa_ref, wb_ref, o_ref):
        m = jnp.dot(h_ref[...], wa_ref[...], preferred_element_type=jnp.float32)
        m = m + jnp.dot(s_ref[...] * scale, wb_ref[...],
                        preferred_element_type=jnp.float32)
        o_ref[:, :h] = h_ref[...]
        o_ref[:, h:] = jnp.maximum(m, 0.0)

    return pl.pallas_call(
        body,
        out_shape=jax.ShapeDtypeStruct(
            (h0.shape[0], h + w_self.shape[1]), jnp.float32),
    )(h0, agg_sum, w_self, w_neigh)


def kernel(feats, node_neigh, W_prep, W0, W1):
    n, s = node_neigh.shape
    p = W_prep.shape[1]
    h0_dim = W0.shape[1]
    scale = 1.0 / s

    npad = -(-n // (_NW * _C)) * (_NW * _C)
    pad_idx = (jnp.arange(s * (npad - n), dtype=jnp.int32)
               .reshape(s, npad - n)) % n
    neigh_t = jnp.concatenate([node_neigh.T, pad_idx], axis=1)
    neigh_c = neigh_t.reshape(s, npad // _C, _C).transpose(1, 0, 2)

    all_feats = _matmul(feats, W_prep)
    s0 = _gather_sum(all_feats, neigh_c, npad)[:n]
    h0 = _layer0(all_feats, s0, W0[:p], W0[p:], scale)
    s1 = _gather_sum(h0, neigh_c, npad)[:n]
    out = _layer1(h0, s1, W1[:h0_dim], W1[h0_dim:], scale)
    return out[None]


# speedup vs baseline: 4.1577x; 1.0264x over previous
import functools

import jax
import jax.numpy as jnp
from jax import lax
from jax.experimental import pallas as pl
from jax.experimental.pallas import tpu as pltpu
from jax.experimental.pallas import tpu_sc as plsc

_NC = 2
_NS = 16
_NW = _NC * _NS
_C = 64


def _gather_sum(table, neigh_c, npad):
    nq, s, c = neigh_c.shape
    d = table.shape[1]
    nch = nq // _NW
    k0 = nq // 2
    mesh = plsc.VectorSubcoreMesh(core_axis_name="c", subcore_axis_name="s")

    @functools.partial(
        pl.kernel,
        out_type=jax.ShapeDtypeStruct((npad, d), jnp.float32),
        mesh=mesh,
        scratch_types=[
            pltpu.VMEM((2, s, _C), jnp.int32),
            pltpu.VMEM((2, _C, d), jnp.float32),
            [pltpu.SemaphoreType.DMA] * 2,
            [pltpu.SemaphoreType.DMA] * 2,
            [pltpu.SemaphoreType.DMA] * 2,
            [pltpu.SemaphoreType.DMA] * 2,
        ],
    )
    def gather_kernel(table_hbm, neigh_hbm, out_hbm, idx_v, acc_v,
                      semI, semF, semA, semO):
        cid = lax.axis_index("c")
        sid = lax.axis_index("s")
        lo = cid * k0 + sid * nch

        def idx_load(i, b, sync):
            cp = pltpu.make_async_copy(neigh_hbm.at[lo + i], idx_v.at[b],
                                       semI[b])
            cp.start()
            if sync:
                cp.wait()

        def first(i, b):
            pltpu.async_copy(table_hbm.at[idx_v.at[b].at[0]], acc_v.at[b],
                             semF[b])

        idx_load(0, 0, True)
        first(0, 0)
        if nch > 1:
            idx_load(1, 1, False)
        pltpu.make_async_copy(table_hbm.at[idx_v.at[0].at[0]], acc_v.at[0],
                              semF[0]).wait()

        for i in range(nch):
            b = i % 2
            o = 1 - b
            def fire(j, cy):
                pltpu.async_copy(table_hbm.at[idx_v.at[b].at[j]],
                                 acc_v.at[b], semA[b], add=True)
                return cy

            with jax.named_scope("fire"):
                lax.fori_loop(1, s, fire, 0, unroll=True)

            if i + 1 < nch:
                pltpu.make_async_copy(neigh_hbm.at[lo + i + 1], idx_v.at[o],
                                      semI[o]).wait()
                if i >= 1:
                    pltpu.make_async_copy(acc_v.at[o],
                                          out_hbm.at[pl.ds((lo + i - 1) * _C,
                                                           _C)],
                                          semO[o]).wait()
                first(i + 1, o)

            def drain(j, cy):
                pltpu.make_async_copy(table_hbm.at[idx_v.at[b].at[0]],
                                      acc_v.at[b], semA[b]).wait()
                return cy

            with jax.named_scope("drain"):
                lax.fori_loop(1, s, drain, 0, unroll=True)

            if i + 2 < nch:
                idx_load(i + 2, b, False)
            pltpu.make_async_copy(acc_v.at[b],
                                  out_hbm.at[pl.ds((lo + i) * _C, _C)],
                                  semO[b]).start()
            if i + 1 < nch:
                pltpu.make_async_copy(table_hbm.at[idx_v.at[o].at[0]],
                                      acc_v.at[o], semF[o]).wait()

        for i in range(max(nch - 2, 0), nch):
            b = i % 2
            pltpu.make_async_copy(acc_v.at[b],
                                  out_hbm.at[pl.ds((lo + i) * _C, _C)],
                                  semO[b]).wait()

    return gather_kernel(table, neigh_c)


_BLK = 1000


def _matmul(x, w):
    n = x.shape[0]

    def body(x_ref, w_ref, o_ref):
        o_ref[...] = jnp.dot(x_ref[...], w_ref[...],
                             preferred_element_type=jnp.float32)

    return pl.pallas_call(
        body,
        grid=(n // _BLK,),
        in_specs=[
            pl.BlockSpec((_BLK, x.shape[1]), lambda i: (i, 0)),
            pl.BlockSpec(w.shape, lambda i: (0, 0)),
        ],
        out_specs=pl.BlockSpec((_BLK, w.shape[1]), lambda i: (i, 0)),
        out_shape=jax.ShapeDtypeStruct((n, w.shape[1]), jnp.float32),
    )(x, w)


def _layer0(x, agg_sum, w_self, w_neigh, scale):
    n, d = x.shape
    h = w_self.shape[1]

    def body(x_ref, s_ref, wa_ref, wb_ref, o_ref):
        m = jnp.dot(x_ref[...], wa_ref[...], preferred_element_type=jnp.float32)
        m = m + jnp.dot(s_ref[...] * scale, wb_ref[...],
                        preferred_element_type=jnp.float32)
        o_ref[...] = jnp.maximum(m, 0.0)

    return pl.pallas_call(
        body,
        grid=(n // _BLK,),
        in_specs=[
            pl.BlockSpec((_BLK, d), lambda i: (i, 0)),
            pl.BlockSpec((_BLK, d), lambda i: (i, 0)),
            pl.BlockSpec(w_self.shape, lambda i: (0, 0)),
            pl.BlockSpec(w_neigh.shape, lambda i: (0, 0)),
        ],
        out_specs=pl.BlockSpec((_BLK, h), lambda i: (i, 0)),
        out_shape=jax.ShapeDtypeStruct((n, h), jnp.float32),
    )(x, agg_sum, w_self, w_neigh)


def _layer1(h0, agg_sum, w_self, w_neigh, scale):
    n, h = h0.shape
    h1 = w_self.shape[1]

    def body(h_ref, s_ref, wa_ref, wb_ref, o_ref):
        m = jnp.dot(h_ref[...], wa_ref[...], preferred_element_type=jnp.float32)
        m = m + jnp.dot(s_ref[...] * scale, wb_ref[...],
                        preferred_element_type=jnp.float32)
        o_ref[0, :, :h] = h_ref[...]
        o_ref[0, :, h:] = jnp.maximum(m, 0.0)

    return pl.pallas_call(
        body,
        grid=(n // _BLK,),
        in_specs=[
            pl.BlockSpec((_BLK, h), lambda i: (i, 0)),
            pl.BlockSpec((_BLK, h), lambda i: (i, 0)),
            pl.BlockSpec(w_self.shape, lambda i: (0, 0)),
            pl.BlockSpec(w_neigh.shape, lambda i: (0, 0)),
        ],
        out_specs=pl.BlockSpec((1, _BLK, h + h1), lambda i: (0, i, 0)),
        out_shape=jax.ShapeDtypeStruct((1, n, h + h1), jnp.float32),
    )(h0, agg_sum, w_self, w_neigh)


def kernel(feats, node_neigh, W_prep, W0, W1):
    n, s = node_neigh.shape
    p = W_prep.shape[1]
    h0_dim = W0.shape[1]
    scale = 1.0 / s

    npad = -(-n // (_NW * _C)) * (_NW * _C)
    pad_idx = (jnp.arange(s * (npad - n), dtype=jnp.int32)
               .reshape(s, npad - n)) % n
    neigh_t = jnp.concatenate([node_neigh.T, pad_idx], axis=1)
    neigh_c = neigh_t.reshape(s, npad // _C, _C).transpose(1, 0, 2)

    all_feats = _matmul(feats, W_prep)
    s0 = _gather_sum(all_feats, neigh_c, npad)
    h0 = _layer0(all_feats, s0, W0[:p], W0[p:], scale)
    s1 = _gather_sum(h0, neigh_c, npad)
    return _layer1(h0, s1, W1[:h0_dim], W1[h0_dim:], scale)


# speedup vs baseline: 4.4022x; 1.0588x over previous
import functools

import jax
import jax.numpy as jnp
from jax import lax
from jax.experimental import pallas as pl
from jax.experimental.pallas import tpu as pltpu
from jax.experimental.pallas import tpu_sc as plsc

_NC = 2
_NS = 16
_NW = _NC * _NS
_C = 64


def _gather_sum(table, neigh_c, npad):
    nq, s, c = neigh_c.shape
    d = table.shape[1]
    nch = nq // _NW
    k0 = nq // 2
    mesh = plsc.VectorSubcoreMesh(core_axis_name="c", subcore_axis_name="s")

    @functools.partial(
        pl.kernel,
        out_type=jax.ShapeDtypeStruct((npad, d), jnp.float32),
        mesh=mesh,
        scratch_types=[
            pltpu.VMEM((2, s, _C), jnp.int32),
            pltpu.VMEM((2, _C, d), jnp.float32),
            [pltpu.SemaphoreType.DMA] * 2,
            [pltpu.SemaphoreType.DMA] * 2,
            [pltpu.SemaphoreType.DMA] * 2,
            [pltpu.SemaphoreType.DMA] * 2,
        ],
    )
    def gather_kernel(table_hbm, neigh_hbm, out_hbm, idx_v, acc_v,
                      semI, semF, semA, semO):
        cid = lax.axis_index("c")
        sid = lax.axis_index("s")
        lo = cid * k0 + sid * nch

        def idx_load(i, b, sync):
            cp = pltpu.make_async_copy(neigh_hbm.at[lo + i], idx_v.at[b],
                                       semI[b])
            cp.start()
            if sync:
                cp.wait()

        def first(i, b):
            pltpu.async_copy(table_hbm.at[idx_v.at[b].at[0]], acc_v.at[b],
                             semF[b])

        idx_load(0, 0, True)
        first(0, 0)
        if nch > 1:
            idx_load(1, 1, False)
        pltpu.make_async_copy(table_hbm.at[idx_v.at[0].at[0]], acc_v.at[0],
                              semF[0]).wait()

        for i in range(nch):
            b = i % 2
            o = 1 - b
            def fire(j, cy):
                pltpu.async_copy(table_hbm.at[idx_v.at[b].at[j]],
                                 acc_v.at[b], semA[b], add=True)
                return cy

            with jax.named_scope("fire"):
                lax.fori_loop(1, s, fire, 0, unroll=True)

            if i + 1 < nch:
                pltpu.make_async_copy(neigh_hbm.at[lo + i + 1], idx_v.at[o],
                                      semI[o]).wait()
                if i >= 1:
                    pltpu.make_async_copy(acc_v.at[o],
                                          out_hbm.at[pl.ds((lo + i - 1) * _C,
                                                           _C)],
                                          semO[o]).wait()
                first(i + 1, o)

            def drain(j, cy):
                pltpu.make_async_copy(table_hbm.at[idx_v.at[b].at[0]],
                                      acc_v.at[b], semA[b]).wait()
                return cy

            with jax.named_scope("drain"):
                lax.fori_loop(1, s, drain, 0, unroll=True)

            if i + 2 < nch:
                idx_load(i + 2, b, False)
            pltpu.make_async_copy(acc_v.at[b],
                                  out_hbm.at[pl.ds((lo + i) * _C, _C)],
                                  semO[b]).start()
            if i + 1 < nch:
                pltpu.make_async_copy(table_hbm.at[idx_v.at[o].at[0]],
                                      acc_v.at[o], semF[o]).wait()

        for i in range(max(nch - 2, 0), nch):
            b = i % 2
            pltpu.make_async_copy(acc_v.at[b],
                                  out_hbm.at[pl.ds((lo + i) * _C, _C)],
                                  semO[b]).wait()

    return gather_kernel(table, neigh_c)


def _matmul(x, w):
    n = x.shape[0]

    def body(x_ref, w_ref, o_ref):
        o_ref[...] = jnp.dot(x_ref[...], w_ref[...],
                             preferred_element_type=jnp.float32)

    return pl.pallas_call(
        body,
        out_shape=jax.ShapeDtypeStruct((n, w.shape[1]), jnp.float32),
    )(x, w)


def _layer0(x, agg_sum, w_self, w_neigh, scale):
    n, d = x.shape
    h = w_self.shape[1]

    def body(x_ref, s_ref, wa_ref, wb_ref, o_ref):
        m = jnp.dot(x_ref[...], wa_ref[...], preferred_element_type=jnp.float32)
        m = m + jnp.dot(s_ref[...] * scale, wb_ref[...],
                        preferred_element_type=jnp.float32)
        o_ref[...] = jnp.maximum(m, 0.0)

    return pl.pallas_call(
        body,
        grid=(1,),
        in_specs=[
            pl.BlockSpec((n, d), lambda i: (0, 0)),
            pl.BlockSpec((n, d), lambda i: (0, 0)),
            pl.BlockSpec(w_self.shape, lambda i: (0, 0)),
            pl.BlockSpec(w_neigh.shape, lambda i: (0, 0)),
        ],
        out_specs=pl.BlockSpec((n, h), lambda i: (0, 0)),
        out_shape=jax.ShapeDtypeStruct((n, h), jnp.float32),
    )(x, agg_sum, w_self, w_neigh)


def _layer1(h0, agg_sum, w_self, w_neigh, scale):
    n, h = h0.shape
    h1 = w_self.shape[1]

    def body(h_ref, s_ref, wa_ref, wb_ref, o_ref):
        m = jnp.dot(h_ref[...], wa_ref[...], preferred_element_type=jnp.float32)
        m = m + jnp.dot(s_ref[...] * scale, wb_ref[...],
                        preferred_element_type=jnp.float32)
        o_ref[0, :, :h] = h_ref[...]
        o_ref[0, :, h:] = jnp.maximum(m, 0.0)

    return pl.pallas_call(
        body,
        grid=(1,),
        in_specs=[
            pl.BlockSpec((n, h), lambda i: (0, 0)),
            pl.BlockSpec((n, h), lambda i: (0, 0)),
            pl.BlockSpec(w_self.shape, lambda i: (0, 0)),
            pl.BlockSpec(w_neigh.shape, lambda i: (0, 0)),
        ],
        out_specs=pl.BlockSpec((1, n, h + h1), lambda i: (0, 0, 0)),
        out_shape=jax.ShapeDtypeStruct((1, n, h + h1), jnp.float32),
    )(h0, agg_sum, w_self, w_neigh)


def kernel(feats, node_neigh, W_prep, W0, W1):
    n, s = node_neigh.shape
    p = W_prep.shape[1]
    h0_dim = W0.shape[1]
    scale = 1.0 / s

    npad = -(-n // (_NW * _C)) * (_NW * _C)
    pad_idx = (jnp.arange(s * (npad - n), dtype=jnp.int32)
               .reshape(s, npad - n)) % n
    neigh_t = jnp.concatenate([node_neigh.T, pad_idx], axis=1)
    neigh_c = neigh_t.reshape(s, npad // _C, _C).transpose(1, 0, 2)

    all_feats = _matmul(feats, W_prep)
    s0 = _gather_sum(all_feats, neigh_c, npad)
    h0 = _layer0(all_feats, s0, W0[:p], W0[p:], scale)
    s1 = _gather_sum(h0, neigh_c, npad)
    return _layer1(h0, s1, W1[:h0_dim], W1[h0_dim:], scale)


# speedup vs baseline: 4.7835x; 1.0866x over previous
import functools

import jax
import jax.numpy as jnp
from jax import lax
from jax.experimental import pallas as pl
from jax.experimental.pallas import tpu as pltpu
from jax.experimental.pallas import tpu_sc as plsc

_NC = 2
_NS = 16
_NW = _NC * _NS
_C = 64


def _gather_sum(table, neigh_c, npad):
    nq, s, c = neigh_c.shape
    d = table.shape[1]
    nch = nq // _NW
    k0 = nq // 2
    mesh = plsc.VectorSubcoreMesh(core_axis_name="c", subcore_axis_name="s")

    @functools.partial(
        pl.kernel,
        out_type=jax.ShapeDtypeStruct((npad, d), jnp.float32),
        mesh=mesh,
        scratch_types=[
            pltpu.VMEM((2, s, _C), jnp.int32),
            pltpu.VMEM((2, _C, d), jnp.float32),
            pltpu.VMEM_SHARED(table.shape, jnp.float32),
            [pltpu.SemaphoreType.DMA] * 2,
            [pltpu.SemaphoreType.DMA] * 2,
            [pltpu.SemaphoreType.DMA] * 2,
            [pltpu.SemaphoreType.DMA] * 2,
        ],
    )
    def gather_kernel(table_hbm, neigh_hbm, out_hbm, idx_v, acc_v, shared_t,
                      semI, semF, semA, semO):
        cid = lax.axis_index("c")
        sid = lax.axis_index("s")
        lo = cid * k0 + sid * nch

        @pl.when(sid == 0)
        def _():
            pltpu.sync_copy(table_hbm, shared_t)

        plsc.subcore_barrier()

        def idx_load(i, b, sync):
            cp = pltpu.make_async_copy(neigh_hbm.at[lo + i], idx_v.at[b],
                                       semI[b])
            cp.start()
            if sync:
                cp.wait()

        def first(i, b):
            pltpu.async_copy(shared_t.at[idx_v.at[b].at[0]], acc_v.at[b],
                             semF[b])

        idx_load(0, 0, True)
        first(0, 0)
        if nch > 1:
            idx_load(1, 1, False)
        pltpu.make_async_copy(table_hbm.at[idx_v.at[0].at[0]], acc_v.at[0],
                              semF[0]).wait()

        for i in range(nch):
            b = i % 2
            o = 1 - b
            def fire(j, cy):
                pltpu.async_copy(shared_t.at[idx_v.at[b].at[j]],
                                 acc_v.at[b], semA[b], add=True)
                return cy

            with jax.named_scope("fire"):
                lax.fori_loop(1, s, fire, 0, unroll=True)

            if i + 1 < nch:
                pltpu.make_async_copy(neigh_hbm.at[lo + i + 1], idx_v.at[o],
                                      semI[o]).wait()
                if i >= 1:
                    pltpu.make_async_copy(acc_v.at[o],
                                          out_hbm.at[pl.ds((lo + i - 1) * _C,
                                                           _C)],
                                          semO[o]).wait()
                first(i + 1, o)

            def drain(j, cy):
                pltpu.make_async_copy(table_hbm.at[idx_v.at[b].at[0]],
                                      acc_v.at[b], semA[b]).wait()
                return cy

            with jax.named_scope("drain"):
                lax.fori_loop(1, s, drain, 0, unroll=True)

            if i + 2 < nch:
                idx_load(i + 2, b, False)
            pltpu.make_async_copy(acc_v.at[b],
                                  out_hbm.at[pl.ds((lo + i) * _C, _C)],
                                  semO[b]).start()
            if i + 1 < nch:
                pltpu.make_async_copy(table_hbm.at[idx_v.at[o].at[0]],
                                      acc_v.at[o], semF[o]).wait()

        for i in range(max(nch - 2, 0), nch):
            b = i % 2
            pltpu.make_async_copy(acc_v.at[b],
                                  out_hbm.at[pl.ds((lo + i) * _C, _C)],
                                  semO[b]).wait()

    return gather_kernel(table, neigh_c)


def _matmul(x, w):
    n = x.shape[0]

    def body(x_ref, w_ref, o_ref):
        o_ref[...] = jnp.dot(x_ref[...], w_ref[...],
                             preferred_element_type=jnp.float32)

    return pl.pallas_call(
        body,
        out_shape=jax.ShapeDtypeStruct((n, w.shape[1]), jnp.float32),
    )(x, w)


def _layer0(x, agg_sum, w_self, w_neigh, scale):
    n, d = x.shape
    h = w_self.shape[1]

    def body(x_ref, s_ref, wa_ref, wb_ref, o_ref):
        m = jnp.dot(x_ref[...], wa_ref[...], preferred_element_type=jnp.float32)
        m = m + jnp.dot(s_ref[...] * scale, wb_ref[...],
                        preferred_element_type=jnp.float32)
        o_ref[...] = jnp.maximum(m, 0.0)

    return pl.pallas_call(
        body,
        grid=(1,),
        in_specs=[
            pl.BlockSpec((n, d), lambda i: (0, 0)),
            pl.BlockSpec((n, d), lambda i: (0, 0)),
            pl.BlockSpec(w_self.shape, lambda i: (0, 0)),
            pl.BlockSpec(w_neigh.shape, lambda i: (0, 0)),
        ],
        out_specs=pl.BlockSpec((n, h), lambda i: (0, 0)),
        out_shape=jax.ShapeDtypeStruct((n, h), jnp.float32),
    )(x, agg_sum, w_self, w_neigh)


def _layer1(h0, agg_sum, w_self, w_neigh, scale):
    n, h = h0.shape
    h1 = w_self.shape[1]

    def body(h_ref, s_ref, wa_ref, wb_ref, o_ref):
        m = jnp.dot(h_ref[...], wa_ref[...], preferred_element_type=jnp.float32)
        m = m + jnp.dot(s_ref[...] * scale, wb_ref[...],
                        preferred_element_type=jnp.float32)
        o_ref[0, :, :h] = h_ref[...]
        o_ref[0, :, h:] = jnp.maximum(m, 0.0)

    return pl.pallas_call(
        body,
        grid=(1,),
        in_specs=[
            pl.BlockSpec((n, h), lambda i: (0, 0)),
            pl.BlockSpec((n, h), lambda i: (0, 0)),
            pl.BlockSpec(w_self.shape, lambda i: (0, 0)),
            pl.BlockSpec(w_neigh.shape, lambda i: (0, 0)),
        ],
        out_specs=pl.BlockSpec((1, n, h + h1), lambda i: (0, 0, 0)),
        out_shape=jax.ShapeDtypeStruct((1, n, h + h1), jnp.float32),
    )(h0, agg_sum, w_self, w_neigh)


def kernel(feats, node_neigh, W_prep, W0, W1):
    n, s = node_neigh.shape
    p = W_prep.shape[1]
    h0_dim = W0.shape[1]
    scale = 1.0 / s

    npad = -(-n // (_NW * _C)) * (_NW * _C)
    pad_idx = (jnp.arange(s * (npad - n), dtype=jnp.int32)
               .reshape(s, npad - n)) % n
    neigh_t = jnp.concatenate([node_neigh.T, pad_idx], axis=1)
    neigh_c = neigh_t.reshape(s, npad // _C, _C).transpose(1, 0, 2)

    all_feats = _matmul(feats, W_prep)
    s0 = _gather_sum(all_feats, neigh_c, npad)
    h0 = _layer0(all_feats, s0, W0[:p], W0[p:], scale)
    s1 = _gather_sum(h0, neigh_c, npad)
    return _layer1(h0, s1, W1[:h0_dim], W1[h0_dim:], scale)
